# baseline (speedup 1.0000x reference)
import functools

import jax
import jax.numpy as jnp
from jax import lax
from jax.experimental import pallas as pl
from jax.experimental.pallas import tpu as pltpu
from jax.experimental.pallas import tpu_sc as plsc

NC = 2
NS = 16
NW = NC * NS
L = 16

CH = 128
NH = 10
TRASH = 8
TR = 128


def _mesh():
    return plsc.VectorSubcoreMesh(core_axis_name="c", subcore_axis_name="s")


def _starts(rows):
    s = list(range(0, rows - CH + 1, CH))
    if s[-1] != rows - CH:
        s.append(rows - CH)
    return s


def _deg_kernel(halfd, nch):
    rpt = halfd // NS

    @functools.partial(
        pl.kernel,
        out_type=jax.ShapeDtypeStruct((NC * NS, rpt, L), jnp.float32),
        mesh=_mesh(),
        scratch_types=[
            pltpu.VMEM((nch, CH), jnp.int32),
            pltpu.VMEM((2, CH), jnp.int32),
            pltpu.VMEM((CH, L), jnp.float32),
            pltpu.VMEM((CH, L), jnp.float32),
            pltpu.VMEM_SHARED((halfd + TRASH, L), jnp.float32),
        ],
    )
    def k(dst_hbm, out_hbm, dst_v, iv, ones_v, wb_v, degm):
        cid = lax.axis_index("c")
        sid = lax.axis_index("s")
        wid = cid * NS + sid
        lane = lax.iota(jnp.int32, L)
        e0 = jnp.where(lane == 0, 1.0, 0.0).astype(jnp.float32)
        zeros = jnp.zeros((L,), jnp.float32)

        def frow(r, _):
            ones_v[r, :] = e0
            wb_v[r, :] = zeros
            return 0

        lax.fori_loop(0, CH, frow, 0)

        base = sid * rpt
        for c in range(CH // L):
            off = jnp.minimum(jnp.full((L,), c * L, jnp.int32) + lane,
                              rpt - 1)
            iv[0, pl.ds(c * L, L)] = base + off
            if c == 0:
                iv[1, pl.ds(0, L)] = jnp.where(
                    lane < TRASH, halfd + lane, halfd
                )
            else:
                iv[1, pl.ds(c * L, L)] = jnp.full((L,), halfd, jnp.int32)

        pltpu.sync_copy(wb_v, degm.at[iv.at[0]])

        @pl.when(sid == 0)
        def _():
            pltpu.sync_copy(wb_v, degm.at[iv.at[1]])

        pltpu.sync_copy(dst_hbm.at[wid], dst_v)
        plsc.subcore_barrier()

        for jj in range(nch):
            pltpu.sync_copy(ones_v, degm.at[dst_v.at[jj]], add=True)
        plsc.subcore_barrier()

        slot = cid * NS + sid
        pltpu.sync_copy(degm.at[iv.at[0]], wb_v)
        pltpu.sync_copy(wb_v.at[pl.ds(0, rpt)], out_hbm.at[slot])

    return k


def _agg_kernel(half, d, nch):
    hps = half // NS
    ndl = d // L
    npl = ndl // 2
    srs = hps * npl
    starts = _starts(srs)
    kr = len(starts)

    @functools.partial(
        pl.kernel,
        out_type=[
            jax.ShapeDtypeStruct((NC * NS, srs, L), jnp.float32),
            jax.ShapeDtypeStruct((NC * NS, srs, L), jnp.float32),
        ],
        mesh=_mesh(),
        scratch_types=[
            pltpu.VMEM((nch, CH), jnp.int32),
            pltpu.VMEM((nch, CH), jnp.int32),
            pltpu.VMEM((1, CH), jnp.int32),
            pltpu.VMEM((ndl, CH), jnp.int32),
            pltpu.VMEM((kr + 1, CH), jnp.int32),
            pltpu.VMEM((CH, d), jnp.float32),
            [pltpu.VMEM((CH, L), jnp.float32) for _ in range(4)],
            pltpu.VMEM((CH, L), jnp.float32),
            pltpu.VMEM_SHARED((half * npl + TRASH * npl, L), jnp.float32),
            pltpu.VMEM_SHARED((half * npl + TRASH * npl, L), jnp.float32),
            pltpu.SemaphoreType.DMA,
        ],
    )
    def k(y_hbm, src_hbm, dst_hbm, out0_hbm, out1_hbm,
          src_v, dst_v, sib, i8, iv, gbuf, sbufs, wb0, acc0, acc1, gsem):
        cid = lax.axis_index("c")
        sid = lax.axis_index("s")
        wid = cid * NS + sid
        lane = lax.iota(jnp.int32, L)
        zeros = jnp.zeros((L,), jnp.float32)
        accs = (acc0, acc1)
        trash = half * npl

        def zrow(r, _):
            wb0[r, :] = zeros
            return 0

        lax.fori_loop(0, CH, zrow, 0)

        base = sid * srs
        for r, st in enumerate(starts):
            for c in range(CH // L):
                iv[r, pl.ds(c * L, L)] = base + st + c * L + lane
        for c in range(CH // L):
            if c < 2:
                iv[kr, pl.ds(c * L, L)] = trash + c * L + lane
            else:
                iv[kr, pl.ds(c * L, L)] = jnp.full((L,), trash, jnp.int32)

        pltpu.sync_copy(src_hbm.at[wid], src_v)
        pltpu.sync_copy(dst_hbm.at[wid], dst_v)

        for a in accs:
            for r in range(kr):
                pltpu.sync_copy(wb0, a.at[iv.at[r]])

            @pl.when(sid == 0)
            def _():
                pltpu.sync_copy(wb0, a.at[iv.at[kr]])

        plsc.subcore_barrier()

        def chunk(jj, _):
            for g in range(CH // L):
                sib[0, pl.ds(g * L, L)] = src_v[jj, pl.ds(g * L, L)]
            pltpu.async_copy(y_hbm.at[sib.at[0]], gbuf, gsem).wait()
            for c in range(ndl):
                for g in range(CH // L):
                    i8[c, pl.ds(g * L, L)] = (
                        dst_v[jj, pl.ds(g * L, L)] * npl + (c % npl)
                    )

            for hc in range(2):
                def trow(r, _):
                    for c in range(npl):
                        sbufs[c][r, :] = gbuf[r, pl.ds((hc * npl + c) * L, L)]
                    return 0

                lax.fori_loop(0, CH, trow, 0)
                for c in range(npl):
                    pltpu.sync_copy(
                        sbufs[c], accs[hc].at[i8.at[hc * npl + c]], add=True,
                    )
            return 0

        lax.fori_loop(0, nch, chunk, 0)
        plsc.subcore_barrier()

        slot = cid * NS + sid
        for a, o_hbm in zip(accs, (out0_hbm, out1_hbm)):
            for r, st in enumerate(starts):
                pltpu.sync_copy(a.at[iv.at[r]], wb0)
                pltpu.sync_copy(wb0, o_hbm.at[slot, pl.ds(st, CH)])

    return k


def _tc1(x_pad, w1, deg_parts, npad, half, d):
    hb = half // TR

    def body(x_ref, w_ref, dp_ref, dis_ref, y_ref):
        deg = dp_ref[0, 0, :, 0] + dp_ref[0, 1, :, 0] + 1.0
        dis = lax.rsqrt(deg)[:, None]
        h = jnp.dot(x_ref[...], w_ref[...], preferred_element_type=jnp.float32)
        dis_ref[...] = dis
        y_ref[...] = h * dis

    return pl.pallas_call(
        body,
        grid=(npad // TR,),
        in_specs=[
            pl.BlockSpec((TR, d), lambda i: (i, 0)),
            pl.BlockSpec((d, d), lambda i: (0, 0)),
            pl.BlockSpec((1, NC, TR, L),
                         lambda i: (i // hb, 0, i % hb, 0)),
        ],
        out_specs=[
            pl.BlockSpec((TR, 1), lambda i: (i, 0)),
            pl.BlockSpec((TR, d), lambda i: (i, 0)),
        ],
        out_shape=[
            jax.ShapeDtypeStruct((npad, 1), jnp.float32),
            jax.ShapeDtypeStruct((npad, d), jnp.float32),
        ],
    )(x_pad, w1, deg_parts)


def _tc2(ap0, ap1, y1, dis, b1, w2, npad, half, d):
    dh = d // 2
    hb = half // TR

    def body(a0_ref, a1_ref, y1_ref, dis_ref, b_ref, w_ref, y2_ref):
        dis = dis_ref[...]
        y1v = y1_ref[...]
        lo = a0_ref[0, 0] + a0_ref[0, 1] + y1v[:, :dh]
        hi = a1_ref[0, 0] + a1_ref[0, 1] + y1v[:, dh:]
        pre = jnp.concatenate([lo, hi], axis=1) * dis + b_ref[...]
        z = jnp.maximum(pre, 0.0)
        y2_ref[...] = (
            jnp.dot(z, w_ref[...], preferred_element_type=jnp.float32) * dis
        )

    return pl.pallas_call(
        body,
        grid=(npad // TR,),
        in_specs=[
            pl.BlockSpec((1, NC, TR, dh), lambda i: (i // hb, 0, i % hb, 0)),
            pl.BlockSpec((1, NC, TR, dh), lambda i: (i // hb, 0, i % hb, 0)),
            pl.BlockSpec((TR, d), lambda i: (i, 0)),
            pl.BlockSpec((TR, 1), lambda i: (i, 0)),
            pl.BlockSpec((1, d), lambda i: (0, 0)),
            pl.BlockSpec((d, d), lambda i: (0, 0)),
        ],
        out_specs=pl.BlockSpec((TR, d), lambda i: (i, 0)),
        out_shape=jax.ShapeDtypeStruct((npad, d), jnp.float32),
    )(ap0, ap1, y1, dis, b1, w2)


def _tc3(ap0, ap1, y2, dis, b2, npad, half, d):
    dh = d // 2
    hb = half // TR

    def body(a0_ref, a1_ref, y2_ref, dis_ref, b_ref, out_ref):
        y2v = y2_ref[...]
        lo = a0_ref[0, 0] + a0_ref[0, 1] + y2v[:, :dh]
        hi = a1_ref[0, 0] + a1_ref[0, 1] + y2v[:, dh:]
        pre = jnp.concatenate([lo, hi], axis=1) * dis_ref[...] + b_ref[...]
        h = jnp.maximum(pre, 0.0)
        m = jnp.max(h, axis=1, keepdims=True)
        e = jnp.exp(h - m)
        out_ref[...] = e / jnp.sum(e, axis=1, keepdims=True)

    return pl.pallas_call(
        body,
        grid=(npad // TR,),
        in_specs=[
            pl.BlockSpec((1, NC, TR, dh), lambda i: (i // hb, 0, i % hb, 0)),
            pl.BlockSpec((1, NC, TR, dh), lambda i: (i // hb, 0, i % hb, 0)),
            pl.BlockSpec((TR, d), lambda i: (i, 0)),
            pl.BlockSpec((TR, 1), lambda i: (i, 0)),
            pl.BlockSpec((1, d), lambda i: (0, 0)),
        ],
        out_specs=pl.BlockSpec((TR, d), lambda i: (i, 0)),
        out_shape=jax.ShapeDtypeStruct((npad, d), jnp.float32),
    )(ap0, ap1, y2, dis, b2)


def kernel(x, edge_index, W1, b1, W2, b2):
    n, d = x.shape
    e = edge_index.shape[1]
    dh = d // 2

    unit = 256
    half = -(-n // (NH * unit)) * unit
    if NH * half - n < TRASH:
        half += unit
    npad = NH * half
    nch = -(-e // (NW * CH))

    src = edge_index[0].astype(jnp.int32)
    dst = edge_index[1].astype(jnp.int32)
    pad = nch * CH * NW - e
    src_p = jnp.concatenate([src, jnp.zeros((pad,), jnp.int32)])
    src3 = src_p.reshape(NW, nch, CH)
    dst_p = jnp.concatenate([dst, jnp.full((pad,), npad - 8, jnp.int32)])
    d3s = []
    for p in range(NH):
        lo = p * half
        inr = (dst_p >= lo) & (dst_p < lo + half)
        d3s.append(jnp.where(inr, dst_p - lo, half).reshape(NW, nch, CH))
    d3 = jnp.stack(d3s)
    x_pad = jnp.pad(x, ((0, npad - n), (0, 0)))
    b1r = b1.reshape(1, d)
    b2r = b2.reshape(1, d)

    nhd = NH * 2
    halfd = npad // nhd
    d3d = []
    for p in range(nhd):
        lo = p * halfd
        inr = (dst_p >= lo) & (dst_p < lo + halfd)
        d3d.append(jnp.where(inr, dst_p - lo, halfd).reshape(NW, nch, CH))
    d3d = jnp.stack(d3d)

    degk = _deg_kernel(halfd, nch)

    def deg_body(c, dd):
        return c, degk(dd)

    _, deg_o = lax.scan(deg_body, 0, d3d)
    deg_parts = deg_o.reshape(nhd, NC, halfd, L)

    dis, y1 = _tc1(x_pad, W1, deg_parts, npad, halfd, d)

    aggk = _agg_kernel(half, d, nch)

    def run_agg(y):
        def body(c, dd):
            o0, o1 = aggk(y, src3, dd)
            return c, (o0, o1)

        _, (o0s, o1s) = lax.scan(body, 0, d3)
        return (o0s.reshape(NH, NC, half, dh), o1s.reshape(NH, NC, half, dh))

    a10, a11 = run_agg(y1)
    y2 = _tc2(a10, a11, y1, dis, b1r, W2, npad, half, d)

    a20, a21 = run_agg(y2)
    out = _tc3(a20, a21, y2, dis, b2r, npad, half, d)
    return out[:n]
